# baseline (speedup 1.0000x reference)
import functools

import jax
import jax.numpy as jnp
from jax import lax
from jax.experimental import pallas as pl
from jax.experimental.pallas import tpu as pltpu
from jax.experimental.pallas import tpu_sc as plsc

B, N, D_IN, D_OUT, K = 2, 4096, 128, 256, 10
TM = 256
NC, NS, L = 2, 16, 16
NW = NC * NS
RPW = (B * N) // NW
NCHUNK = N // L


def _s1_body(ptsT_ref, ptsN_ref, f_ref, w1a_ref, w1b_ref, d2_ref, h1_ref):
    t = pl.program_id(1)
    PT = ptsT_ref[0]
    Xr, Yr, Zr, Mr = PT[0:1, :], PT[1:2, :], PT[2:3, :], PT[3:4, :]
    msum = jnp.sum(Mr, keepdims=True)
    tile = ptsN_ref[0, pl.ds(t * TM, TM), :]
    xi, yi, zi = tile[:, 0:1], tile[:, 1:2], tile[:, 2:3]
    dx = xi - Xr
    dy = yi - Yr
    dz = zi - Zr
    d2 = (dx * dx + dy * dy) + dz * dz
    d2_ref[0] = d2
    density = jnp.sum(
        jnp.where((d2 < (0.02 ** 2)) & (Mr > 0.0), 1.0, 0.0),
        axis=1, keepdims=True)
    gate = (msum > 0.0).astype(jnp.float32)
    h = jnp.dot(f_ref[0], w1a_ref[...], preferred_element_type=jnp.float32)
    h1_ref[0] = h + (density * gate) * w1b_ref[3:4, :]


_mesh = plsc.VectorSubcoreMesh(core_axis_name="c", subcore_axis_name="s")


@functools.partial(
    pl.kernel,
    out_type=[jax.ShapeDtypeStruct((B * N, L), jnp.float32)] * 3,
    mesh=_mesh,
    compiler_params=pltpu.CompilerParams(needs_layout_passes=False),
    scratch_types=[
        pltpu.VMEM((N,), jnp.float32),
        pltpu.VMEM((N,), jnp.float32),
        pltpu.VMEM((N,), jnp.float32),
        pltpu.VMEM((N,), jnp.float32),
        pltpu.VMEM((N,), jnp.float32),
        pltpu.VMEM((RPW, L), jnp.float32),
        pltpu.VMEM((RPW, L), jnp.float32),
        pltpu.VMEM((RPW, L), jnp.float32),
        pltpu.SemaphoreType.DMA,
        pltpu.SemaphoreType.DMA,
    ],
)
def _sc_topk(d2_hbm, px_hbm, py_hbm, pz_hbm, nx_hbm, ny_hbm, nz_hbm,
             xv, yv, zv, rowa, rowb, nxs, nys, nzs, sema, semb):
    cid = lax.axis_index("c")
    sid = lax.axis_index("s")
    w = sid * NC + cid
    g0 = w * RPW
    b = g0 // N
    pltpu.sync_copy(px_hbm.at[pl.ds(b * N, N)], xv)
    pltpu.sync_copy(py_hbm.at[pl.ds(b * N, N)], yv)
    pltpu.sync_copy(pz_hbm.at[pl.ds(b * N, N)], zv)

    iota = lax.broadcasted_iota(jnp.int32, (L,), 0)

    def do_row(row_ref, r):
        v0 = row_ref[0:L]
        bk, bi = plsc.sort_key_val(v0, iota)
        thr = jnp.full((L,), jnp.max(bk), jnp.float32)

        def chunk_fn(c, carry):
            bk, bi, thr = carry
            v = row_ref[pl.ds(c * L, L)]
            msk = v < thr

            def merge(_):
                si = c * L + iota
                sk, sv = plsc.sort_key_val(v, si)
                rk = lax.rev(sk, (0,))
                rv = lax.rev(sv, (0,))
                take_b = bk <= rk
                mk = jnp.where(take_b, bk, rk)
                mi = jnp.where(take_b, bi, rv)
                nbk, nbi = plsc.sort_key_val(mk, mi)
                nthr = jnp.full((L,), jnp.max(mk), jnp.float32)
                return nbk, nbi, nthr

            def keep(_):
                return bk, bi, thr

            return lax.cond(jnp.any(msk), merge, keep, 0)

        bk, bi, thr = lax.fori_loop(1, NCHUNK, chunk_fn, (bk, bi, thr))
        nxs[r, :] = plsc.load_gather(xv, [bi])
        nys[r, :] = plsc.load_gather(yv, [bi])
        nzs[r, :] = plsc.load_gather(zv, [bi])

    pltpu.async_copy(d2_hbm.at[g0], rowa, sema)

    def pair_fn(p, carry):
        g = g0 + 2 * p
        pltpu.async_copy(d2_hbm.at[g + 1], rowb, semb)
        pltpu.make_async_copy(d2_hbm.at[g], rowa, sema).wait()
        do_row(rowa, 2 * p)

        @pl.when(p < RPW // 2 - 1)
        def _():
            pltpu.async_copy(d2_hbm.at[g + 2], rowa, sema)

        pltpu.make_async_copy(d2_hbm.at[g + 1], rowb, semb).wait()
        do_row(rowb, 2 * p + 1)
        return carry

    lax.fori_loop(0, RPW // 2, pair_fn, 0)

    pltpu.sync_copy(nxs, nx_hbm.at[pl.ds(g0, RPW)])
    pltpu.sync_copy(nys, ny_hbm.at[pl.ds(g0, RPW)])
    pltpu.sync_copy(nzs, nz_hbm.at[pl.ds(g0, RPW)])


def _s3_body(ptsT_ref, ptsN_ref, nx_ref, ny_ref, nz_ref, h1_ref, w1b_ref,
             b1_ref, w2_ref, b2_ref, o_ref):
    t = pl.program_id(1)
    PT = ptsT_ref[0]
    Xr, Yr, Zr, Mr = PT[0:1, :], PT[1:2, :], PT[2:3, :], PT[3:4, :]
    msum = jnp.sum(Mr, keepdims=True)
    denom = jnp.maximum(msum, 1.0)
    cx = jnp.sum(Xr * Mr, keepdims=True) / denom
    cy = jnp.sum(Yr * Mr, keepdims=True) / denom
    cz = jnp.sum(Zr * Mr, keepdims=True) / denom

    tile = ptsN_ref[0, pl.ds(t * TM, TM), :]
    xi, yi, zi = tile[:, 0:1], tile[:, 1:2], tile[:, 2:3]

    lanes = lax.broadcasted_iota(jnp.int32, (1, L), 1)
    kmask = (lanes < K).astype(jnp.float32)
    dxk = (nx_ref[0] - xi) * kmask
    dyk = (ny_ref[0] - yi) * kmask
    dzk = (nz_ref[0] - zi) * kmask
    Kf = float(K)
    cxx = jnp.sum(dxk * dxk, axis=1, keepdims=True) / Kf
    cyy = jnp.sum(dyk * dyk, axis=1, keepdims=True) / Kf
    czz = jnp.sum(dzk * dzk, axis=1, keepdims=True) / Kf
    cxy = jnp.sum(dxk * dyk, axis=1, keepdims=True) / Kf
    cxz = jnp.sum(dxk * dzk, axis=1, keepdims=True) / Kf
    cyz = jnp.sum(dyk * dzk, axis=1, keepdims=True) / Kf

    q = (cxx + cyy + czz) / 3.0
    p1 = cxy * cxy + cxz * cxz + cyz * cyz
    dxx = cxx - q
    dyy = cyy - q
    dzz = czz - q
    p2 = dxx * dxx + dyy * dyy + dzz * dzz + 2.0 * p1
    degen = p2 <= 1e-22
    p = jnp.sqrt(jnp.maximum(p2, 1e-22) / 6.0)
    bxx = dxx / p
    byy = dyy / p
    bzz = dzz / p
    bxy = cxy / p
    bxz = cxz / p
    byz = cyz / p
    detb = (bxx * (byy * bzz - byz * byz)
            - bxy * (bxy * bzz - byz * bxz)
            + bxz * (bxy * byz - byy * bxz))
    r = jnp.clip(detb * 0.5, -1.0, 1.0)
    phi = jnp.arctan2(jnp.sqrt(jnp.maximum(1.0 - r * r, 0.0)), r) / 3.0
    ev2 = q + 2.0 * p * jnp.cos(phi)
    ev0 = q + 2.0 * p * jnp.cos(phi + 2.0 * jnp.pi / 3.0)
    ev2 = jnp.where(degen, q, ev2)
    ev0 = jnp.where(degen, q, ev0)
    curv = ev0 / (ev2 + 1e-8)

    dxc = xi - cx
    dyc = yi - cy
    dzc = zi - cz
    dist_c = jnp.sqrt(dxc * dxc + dyc * dyc + dzc * dzc)
    hdist = jnp.sqrt(dxc * dxc + dyc * dyc)
    rad = jnp.arctan2(dyc, dxc)

    gate = (msum > 0.0).astype(jnp.float32)
    w1b = w1b_ref[...]
    gcon = (dist_c * w1b[0:1, :] + dzc * w1b[1:2, :]
            + hdist * w1b[2:3, :] + curv * w1b[4:5, :]
            + rad * w1b[5:6, :]) * gate

    h = jnp.maximum(h1_ref[0] + gcon + b1_ref[...], 0.0)
    out = jnp.dot(h, w2_ref[...], preferred_element_type=jnp.float32)
    o_ref[0] = jnp.maximum(out + b2_ref[...], 0.0)


@jax.jit
def kernel(points, features, leaf_mask, W1, b1, W2, b2):
    maskf = leaf_mask.astype(jnp.float32)
    ptsT = jnp.concatenate(
        [points.transpose(0, 2, 1), maskf[:, None, :],
         jnp.zeros((B, 4, N), jnp.float32)], axis=1)
    ptsN = jnp.pad(points, ((0, 0), (0, 0), (0, 5)))
    w1a = W1[:D_IN, :]
    w1b = jnp.pad(W1[D_IN:, :], ((0, 2), (0, 0)))
    b1r = b1[None, :]
    b2r = b2[None, :]

    grid = (B, N // TM)
    d2, h1 = pl.pallas_call(
        _s1_body,
        grid=grid,
        in_specs=[
            pl.BlockSpec((1, 8, N), lambda b, t: (b, 0, 0)),
            pl.BlockSpec((1, N, 8), lambda b, t: (b, 0, 0)),
            pl.BlockSpec((1, TM, D_IN), lambda b, t: (b, t, 0)),
            pl.BlockSpec((D_IN, D_OUT), lambda b, t: (0, 0)),
            pl.BlockSpec((8, D_OUT), lambda b, t: (0, 0)),
        ],
        out_specs=[
            pl.BlockSpec((1, TM, N), lambda b, t: (b, t, 0)),
            pl.BlockSpec((1, TM, D_OUT), lambda b, t: (b, t, 0)),
        ],
        out_shape=[
            jax.ShapeDtypeStruct((B, N, N), jnp.float32),
            jax.ShapeDtypeStruct((B, N, D_OUT), jnp.float32),
        ],
    )(ptsT, ptsN, features, w1a, w1b)

    d2f = d2.reshape(B * N, N)
    px = points[..., 0].reshape(B * N)
    py = points[..., 1].reshape(B * N)
    pz = points[..., 2].reshape(B * N)
    nx, ny, nz = _sc_topk(d2f, px, py, pz)
    nx = nx.reshape(B, N, L)
    ny = ny.reshape(B, N, L)
    nz = nz.reshape(B, N, L)

    return pl.pallas_call(
        _s3_body,
        grid=grid,
        in_specs=[
            pl.BlockSpec((1, 8, N), lambda b, t: (b, 0, 0)),
            pl.BlockSpec((1, N, 8), lambda b, t: (b, 0, 0)),
            pl.BlockSpec((1, TM, L), lambda b, t: (b, t, 0)),
            pl.BlockSpec((1, TM, L), lambda b, t: (b, t, 0)),
            pl.BlockSpec((1, TM, L), lambda b, t: (b, t, 0)),
            pl.BlockSpec((1, TM, D_OUT), lambda b, t: (b, t, 0)),
            pl.BlockSpec((8, D_OUT), lambda b, t: (0, 0)),
            pl.BlockSpec((1, D_OUT), lambda b, t: (0, 0)),
            pl.BlockSpec((D_OUT, D_OUT), lambda b, t: (0, 0)),
            pl.BlockSpec((1, D_OUT), lambda b, t: (0, 0)),
        ],
        out_specs=pl.BlockSpec((1, TM, D_OUT), lambda b, t: (b, t, 0)),
        out_shape=jax.ShapeDtypeStruct((B, N, D_OUT), jnp.float32),
    )(ptsT, ptsN, nx, ny, nz, h1, w1b, b1r, W2, b2r)


# speedup vs baseline: 2.7515x; 2.7515x over previous
import functools

import jax
import jax.numpy as jnp
from jax import lax
from jax.experimental import pallas as pl
from jax.experimental.pallas import tpu as pltpu
from jax.experimental.pallas import tpu_sc as plsc

B, N, D_IN, D_OUT, K = 2, 4096, 128, 256, 10
TM = 256
NC, NS, L = 2, 16, 16
NW = NC * NS
RPW = (B * N) // NW
NCHUNK = N // L


def _s1_body(ptsT_ref, ptsN_ref, f_ref, w1a_ref, w1b_ref, d2_ref, h1_ref):
    t = pl.program_id(1)
    PT = ptsT_ref[0]
    Xr, Yr, Zr, Mr = PT[0:1, :], PT[1:2, :], PT[2:3, :], PT[3:4, :]
    msum = jnp.sum(Mr, keepdims=True)
    tile = ptsN_ref[0, pl.ds(t * TM, TM), :]
    xi, yi, zi = tile[:, 0:1], tile[:, 1:2], tile[:, 2:3]
    dx = xi - Xr
    dy = yi - Yr
    dz = zi - Zr
    d2 = (dx * dx + dy * dy) + dz * dz
    d2_ref[0] = d2
    density = jnp.sum(
        jnp.where((d2 < (0.02 ** 2)) & (Mr > 0.0), 1.0, 0.0),
        axis=1, keepdims=True)
    gate = (msum > 0.0).astype(jnp.float32)
    h = jnp.dot(f_ref[0], w1a_ref[...], preferred_element_type=jnp.float32)
    h1_ref[0] = h + (density * gate) * w1b_ref[3:4, :]


_mesh = plsc.VectorSubcoreMesh(core_axis_name="c", subcore_axis_name="s")


RPD = 8
SB = 2 * RPD
GC = 8
_INF = 3e38


@functools.partial(
    pl.kernel,
    out_type=[jax.ShapeDtypeStruct((B * N, 128), jnp.float32)] * 3,
    mesh=_mesh,
    compiler_params=pltpu.CompilerParams(needs_layout_passes=False),
    scratch_types=[
        pltpu.VMEM((N,), jnp.float32),
        pltpu.VMEM((N,), jnp.float32),
        pltpu.VMEM((N,), jnp.float32),
        pltpu.VMEM((RPD, N), jnp.float32),
        pltpu.VMEM((RPD, N), jnp.float32),
        pltpu.VMEM((SB, 128), jnp.float32),
        pltpu.VMEM((SB, 128), jnp.float32),
        pltpu.VMEM((SB, 128), jnp.float32),
        pltpu.SemaphoreType.DMA,
        pltpu.SemaphoreType.DMA,
    ],
)
def _sc_topk(d2_hbm, px_hbm, py_hbm, pz_hbm, nx_hbm, ny_hbm, nz_hbm,
             xv, yv, zv, rowa, rowb, nxs, nys, nzs, sema, semb):
    cid = lax.axis_index("c")
    sid = lax.axis_index("s")
    w = sid * NC + cid
    g0 = w * RPW
    b = g0 // N
    pltpu.sync_copy(px_hbm.at[pl.ds(b * N, N)], xv)
    pltpu.sync_copy(py_hbm.at[pl.ds(b * N, N)], yv)
    pltpu.sync_copy(pz_hbm.at[pl.ds(b * N, N)], zv)

    iota = lax.broadcasted_iota(jnp.int32, (L,), 0)
    mask10 = iota < K

    def do_row(buf, j, r):
        def mn_fn(c, mn):
            for u in range(16):
                mn = jnp.minimum(mn, buf[j, pl.ds((c * 16 + u) * L, L)])
            return mn
        mn = lax.fori_loop(0, NCHUNK // 16, mn_fn,
                           jnp.full((L,), _INF, jnp.float32))
        smn, _ = plsc.sort_key_val(mn, iota)
        t = jnp.max(jnp.where(mask10, smn, -_INF))
        thr = jnp.full((L,), t, jnp.float32)

        def grp_fn(g, carry):
            base = g * (GC * L)
            m_or = None
            for u in range(GC):
                m = buf[j, pl.ds(base + u * L, L)] <= thr
                m_or = m if m_or is None else (m_or | m)

            def hit(c1):
                bk, bi = c1
                for u in range(GC):
                    v = buf[j, pl.ds(base + u * L, L)]

                    def merge(c2, v=v, u=u):
                        bk2, bi2 = c2
                        si = base + u * L + iota
                        sk, sv = plsc.sort_key_val(v, si)
                        rk = lax.rev(sk, (0,))
                        rv = lax.rev(sv, (0,))
                        tb = bk2 <= rk
                        mk = jnp.where(tb, bk2, rk)
                        mi = jnp.where(tb, bi2, rv)
                        nk, ni = plsc.sort_key_val(mk, mi)
                        return (nk, ni)

                    bk, bi = lax.cond(jnp.any(v <= thr), merge,
                                      lambda c2: c2, (bk, bi))
                return bk, bi

            return lax.cond(jnp.any(m_or), hit, lambda c1: c1, carry)

        bk, bi = lax.fori_loop(
            0, NCHUNK // GC, grp_fn,
            (jnp.full((L,), _INF, jnp.float32), jnp.zeros((L,), jnp.int32)))
        nxs[r, 0:L] = plsc.load_gather(xv, [bi])
        nys[r, 0:L] = plsc.load_gather(yv, [bi])
        nzs[r, 0:L] = plsc.load_gather(zv, [bi])

    pltpu.async_copy(d2_hbm.at[pl.ds(g0, RPD)], rowa, sema)

    def pair_fn(p, carry):
        g = g0 + SB * p
        pltpu.async_copy(d2_hbm.at[pl.ds(g + RPD, RPD)], rowb, semb)
        pltpu.make_async_copy(d2_hbm.at[pl.ds(g, RPD)], rowa, sema).wait()
        for j in range(RPD):
            do_row(rowa, j, j)

        @pl.when(p < RPW // SB - 1)
        def _():
            pltpu.async_copy(d2_hbm.at[pl.ds(g + SB, RPD)], rowa, sema)

        pltpu.make_async_copy(d2_hbm.at[pl.ds(g + RPD, RPD)], rowb, semb).wait()
        for j in range(RPD):
            do_row(rowb, j, RPD + j)
        pltpu.sync_copy(nxs, nx_hbm.at[pl.ds(g, SB)])
        pltpu.sync_copy(nys, ny_hbm.at[pl.ds(g, SB)])
        pltpu.sync_copy(nzs, nz_hbm.at[pl.ds(g, SB)])
        return carry

    lax.fori_loop(0, RPW // SB, pair_fn, 0)


def _s3_body(ptsT_ref, ptsN_ref, nx_ref, ny_ref, nz_ref, h1_ref, w1b_ref,
             b1_ref, w2_ref, b2_ref, o_ref):
    t = pl.program_id(1)
    PT = ptsT_ref[0]
    Xr, Yr, Zr, Mr = PT[0:1, :], PT[1:2, :], PT[2:3, :], PT[3:4, :]
    msum = jnp.sum(Mr, keepdims=True)
    denom = jnp.maximum(msum, 1.0)
    cx = jnp.sum(Xr * Mr, keepdims=True) / denom
    cy = jnp.sum(Yr * Mr, keepdims=True) / denom
    cz = jnp.sum(Zr * Mr, keepdims=True) / denom

    tile = ptsN_ref[0, pl.ds(t * TM, TM), :]
    xi, yi, zi = tile[:, 0:1], tile[:, 1:2], tile[:, 2:3]

    lanes = lax.broadcasted_iota(jnp.int32, (1, 128), 1)
    kmask = lanes < K
    dxk = jnp.where(kmask, nx_ref[0] - xi, 0.0)
    dyk = jnp.where(kmask, ny_ref[0] - yi, 0.0)
    dzk = jnp.where(kmask, nz_ref[0] - zi, 0.0)
    Kf = float(K)
    cxx = jnp.sum(dxk * dxk, axis=1, keepdims=True) / Kf
    cyy = jnp.sum(dyk * dyk, axis=1, keepdims=True) / Kf
    czz = jnp.sum(dzk * dzk, axis=1, keepdims=True) / Kf
    cxy = jnp.sum(dxk * dyk, axis=1, keepdims=True) / Kf
    cxz = jnp.sum(dxk * dzk, axis=1, keepdims=True) / Kf
    cyz = jnp.sum(dyk * dzk, axis=1, keepdims=True) / Kf

    q = (cxx + cyy + czz) / 3.0
    p1 = cxy * cxy + cxz * cxz + cyz * cyz
    dxx = cxx - q
    dyy = cyy - q
    dzz = czz - q
    p2 = dxx * dxx + dyy * dyy + dzz * dzz + 2.0 * p1
    degen = p2 <= 1e-22
    p = jnp.sqrt(jnp.maximum(p2, 1e-22) / 6.0)
    bxx = dxx / p
    byy = dyy / p
    bzz = dzz / p
    bxy = cxy / p
    bxz = cxz / p
    byz = cyz / p
    detb = (bxx * (byy * bzz - byz * byz)
            - bxy * (bxy * bzz - byz * bxz)
            + bxz * (bxy * byz - byy * bxz))
    r = jnp.clip(detb * 0.5, -1.0, 1.0)
    phi = jnp.arctan2(jnp.sqrt(jnp.maximum(1.0 - r * r, 0.0)), r) / 3.0
    ev2 = q + 2.0 * p * jnp.cos(phi)
    ev0 = q + 2.0 * p * jnp.cos(phi + 2.0 * jnp.pi / 3.0)
    ev2 = jnp.where(degen, q, ev2)
    ev0 = jnp.where(degen, q, ev0)
    curv = ev0 / (ev2 + 1e-8)

    dxc = xi - cx
    dyc = yi - cy
    dzc = zi - cz
    dist_c = jnp.sqrt(dxc * dxc + dyc * dyc + dzc * dzc)
    hdist = jnp.sqrt(dxc * dxc + dyc * dyc)
    rad = jnp.arctan2(dyc, dxc)

    gate = (msum > 0.0).astype(jnp.float32)
    w1b = w1b_ref[...]
    gcon = (dist_c * w1b[0:1, :] + dzc * w1b[1:2, :]
            + hdist * w1b[2:3, :] + curv * w1b[4:5, :]
            + rad * w1b[5:6, :]) * gate

    h = jnp.maximum(h1_ref[0] + gcon + b1_ref[...], 0.0)
    out = jnp.dot(h, w2_ref[...], preferred_element_type=jnp.float32)
    o_ref[0] = jnp.maximum(out + b2_ref[...], 0.0)


@jax.jit
def kernel(points, features, leaf_mask, W1, b1, W2, b2):
    maskf = leaf_mask.astype(jnp.float32)
    ptsT = jnp.concatenate(
        [points.transpose(0, 2, 1), maskf[:, None, :],
         jnp.zeros((B, 4, N), jnp.float32)], axis=1)
    ptsN = jnp.pad(points, ((0, 0), (0, 0), (0, 5)))
    w1a = W1[:D_IN, :]
    w1b = jnp.pad(W1[D_IN:, :], ((0, 2), (0, 0)))
    b1r = b1[None, :]
    b2r = b2[None, :]

    grid = (B, N // TM)
    d2, h1 = pl.pallas_call(
        _s1_body,
        grid=grid,
        in_specs=[
            pl.BlockSpec((1, 8, N), lambda b, t: (b, 0, 0)),
            pl.BlockSpec((1, N, 8), lambda b, t: (b, 0, 0)),
            pl.BlockSpec((1, TM, D_IN), lambda b, t: (b, t, 0)),
            pl.BlockSpec((D_IN, D_OUT), lambda b, t: (0, 0)),
            pl.BlockSpec((8, D_OUT), lambda b, t: (0, 0)),
        ],
        out_specs=[
            pl.BlockSpec((1, TM, N), lambda b, t: (b, t, 0)),
            pl.BlockSpec((1, TM, D_OUT), lambda b, t: (b, t, 0)),
        ],
        out_shape=[
            jax.ShapeDtypeStruct((B, N, N), jnp.float32),
            jax.ShapeDtypeStruct((B, N, D_OUT), jnp.float32),
        ],
    )(ptsT, ptsN, features, w1a, w1b)

    d2f = d2.reshape(B * N, N)
    px = points[..., 0].reshape(B * N)
    py = points[..., 1].reshape(B * N)
    pz = points[..., 2].reshape(B * N)
    nx, ny, nz = _sc_topk(d2f, px, py, pz)
    nx = nx.reshape(B, N, 128)
    ny = ny.reshape(B, N, 128)
    nz = nz.reshape(B, N, 128)

    return pl.pallas_call(
        _s3_body,
        grid=grid,
        in_specs=[
            pl.BlockSpec((1, 8, N), lambda b, t: (b, 0, 0)),
            pl.BlockSpec((1, N, 8), lambda b, t: (b, 0, 0)),
            pl.BlockSpec((1, TM, 128), lambda b, t: (b, t, 0)),
            pl.BlockSpec((1, TM, 128), lambda b, t: (b, t, 0)),
            pl.BlockSpec((1, TM, 128), lambda b, t: (b, t, 0)),
            pl.BlockSpec((1, TM, D_OUT), lambda b, t: (b, t, 0)),
            pl.BlockSpec((8, D_OUT), lambda b, t: (0, 0)),
            pl.BlockSpec((1, D_OUT), lambda b, t: (0, 0)),
            pl.BlockSpec((D_OUT, D_OUT), lambda b, t: (0, 0)),
            pl.BlockSpec((1, D_OUT), lambda b, t: (0, 0)),
        ],
        out_specs=pl.BlockSpec((1, TM, D_OUT), lambda b, t: (b, t, 0)),
        out_shape=jax.ShapeDtypeStruct((B, N, D_OUT), jnp.float32),
    )(ptsT, ptsN, nx, ny, nz, h1, w1b, b1r, W2, b2r)


# speedup vs baseline: 3.2090x; 1.1663x over previous
import functools

import jax
import jax.numpy as jnp
from jax import lax
from jax.experimental import pallas as pl
from jax.experimental.pallas import tpu as pltpu
from jax.experimental.pallas import tpu_sc as plsc

B, N, D_IN, D_OUT, K = 2, 4096, 128, 256, 10
TM = 256
NC, NS, L = 2, 16, 16
NW = NC * NS
RPW = (B * N) // NW
NCHUNK = N // L


def _s1_body(ptsT_ref, ptsN_ref, f_ref, w1a_ref, w1b_ref, d2_ref, h1_ref,
             thr_ref):
    t = pl.program_id(1)
    PT = ptsT_ref[0]
    Xr, Yr, Zr, Mr = PT[0:1, :], PT[1:2, :], PT[2:3, :], PT[3:4, :]
    msum = jnp.sum(Mr, keepdims=True)
    tile = ptsN_ref[0, pl.ds(t * TM, TM), :]
    xi, yi, zi = tile[:, 0:1], tile[:, 1:2], tile[:, 2:3]
    dx = xi - Xr
    dy = yi - Yr
    dz = zi - Zr
    d2 = (dx * dx + dy * dy) + dz * dz
    d2_ref[0] = d2
    density = jnp.sum(
        jnp.where((d2 < (0.02 ** 2)) & (Mr > 0.0), 1.0, 0.0),
        axis=1, keepdims=True)
    gate = (msum > 0.0).astype(jnp.float32)
    h = jnp.dot(f_ref[0], w1a_ref[...], preferred_element_type=jnp.float32)
    h1_ref[0] = h + (density * gate) * w1b_ref[3:4, :]
    bm = d2[:, 0:128]
    for kb in range(1, N // 128):
        bm = jnp.minimum(bm, d2[:, kb * 128:(kb + 1) * 128])
    m = jnp.min(bm, axis=1, keepdims=True)
    for _ in range(K - 1):
        bm = jnp.where(bm == m, 1e30, bm)
        m = jnp.min(bm, axis=1, keepdims=True)
    thr_ref[0] = m


_mesh = plsc.VectorSubcoreMesh(core_axis_name="c", subcore_axis_name="s")


RPD = 8
SB = 2 * RPD
GC = 8
_INF = 3e38


@functools.partial(
    pl.kernel,
    out_type=[jax.ShapeDtypeStruct((B * N, 128), jnp.float32)] * 3,
    mesh=_mesh,
    compiler_params=pltpu.CompilerParams(needs_layout_passes=False),
    scratch_types=[
        pltpu.VMEM((N,), jnp.float32),
        pltpu.VMEM((N,), jnp.float32),
        pltpu.VMEM((N,), jnp.float32),
        pltpu.VMEM((RPD, N), jnp.float32),
        pltpu.VMEM((RPD, N), jnp.float32),
        pltpu.VMEM((SB, 128), jnp.float32),
        pltpu.VMEM((SB, 128), jnp.float32),
        pltpu.VMEM((SB, 128), jnp.float32),
        pltpu.VMEM((RPW,), jnp.float32),
        pltpu.SemaphoreType.DMA,
        pltpu.SemaphoreType.DMA,
    ],
)
def _sc_topk(d2_hbm, px_hbm, py_hbm, pz_hbm, thr_hbm, nx_hbm, ny_hbm, nz_hbm,
             xv, yv, zv, rowa, rowb, nxs, nys, nzs, thrv, sema, semb):
    cid = lax.axis_index("c")
    sid = lax.axis_index("s")
    w = sid * NC + cid
    g0 = w * RPW
    b = g0 // N
    pltpu.sync_copy(px_hbm.at[pl.ds(b * N, N)], xv)
    pltpu.sync_copy(py_hbm.at[pl.ds(b * N, N)], yv)
    pltpu.sync_copy(pz_hbm.at[pl.ds(b * N, N)], zv)
    pltpu.sync_copy(thr_hbm.at[pl.ds(g0, RPW)], thrv)

    iota = lax.broadcasted_iota(jnp.int32, (L,), 0)

    def do_row(buf, j, r, ts):
        thr = jnp.full((L,), ts, jnp.float32)

        def grp_fn(g, carry):
            base = g * (GC * L)
            m_or = None
            for u in range(GC):
                m = buf[j, pl.ds(base + u * L, L)] <= thr
                m_or = m if m_or is None else (m_or | m)

            def hit(c1):
                bk, bi = c1
                for u in range(GC):
                    v = buf[j, pl.ds(base + u * L, L)]

                    def merge(c2, v=v, u=u):
                        bk2, bi2 = c2
                        si = base + u * L + iota
                        sk, sv = plsc.sort_key_val(v, si)
                        rk = lax.rev(sk, (0,))
                        rv = lax.rev(sv, (0,))
                        tb = bk2 <= rk
                        mk = jnp.where(tb, bk2, rk)
                        mi = jnp.where(tb, bi2, rv)
                        nk, ni = plsc.sort_key_val(mk, mi)
                        return (nk, ni)

                    bk, bi = lax.cond(jnp.any(v <= thr), merge,
                                      lambda c2: c2, (bk, bi))
                return bk, bi

            return lax.cond(jnp.any(m_or), hit, lambda c1: c1, carry)

        bk, bi = lax.fori_loop(
            0, NCHUNK // GC, grp_fn,
            (jnp.full((L,), _INF, jnp.float32), jnp.zeros((L,), jnp.int32)))
        nxs[r, 0:L] = plsc.load_gather(xv, [bi])
        nys[r, 0:L] = plsc.load_gather(yv, [bi])
        nzs[r, 0:L] = plsc.load_gather(zv, [bi])

    pltpu.async_copy(d2_hbm.at[pl.ds(g0, RPD)], rowa, sema)

    def pair_fn(p, carry):
        g = g0 + SB * p
        tch = thrv[pl.ds(SB * p, L)]
        pltpu.async_copy(d2_hbm.at[pl.ds(g + RPD, RPD)], rowb, semb)
        pltpu.make_async_copy(d2_hbm.at[pl.ds(g, RPD)], rowa, sema).wait()
        for j in range(RPD):
            do_row(rowa, j, j, tch[j])

        @pl.when(p < RPW // SB - 1)
        def _():
            pltpu.async_copy(d2_hbm.at[pl.ds(g + SB, RPD)], rowa, sema)

        pltpu.make_async_copy(d2_hbm.at[pl.ds(g + RPD, RPD)], rowb, semb).wait()
        for j in range(RPD):
            do_row(rowb, j, RPD + j, tch[RPD + j])
        pltpu.sync_copy(nxs, nx_hbm.at[pl.ds(g, SB)])
        pltpu.sync_copy(nys, ny_hbm.at[pl.ds(g, SB)])
        pltpu.sync_copy(nzs, nz_hbm.at[pl.ds(g, SB)])
        return carry

    lax.fori_loop(0, RPW // SB, pair_fn, 0)


def _s3_body(ptsT_ref, ptsN_ref, nx_ref, ny_ref, nz_ref, h1_ref, w1b_ref,
             b1_ref, w2_ref, b2_ref, o_ref):
    t = pl.program_id(1)
    PT = ptsT_ref[0]
    Xr, Yr, Zr, Mr = PT[0:1, :], PT[1:2, :], PT[2:3, :], PT[3:4, :]
    msum = jnp.sum(Mr, keepdims=True)
    denom = jnp.maximum(msum, 1.0)
    cx = jnp.sum(Xr * Mr, keepdims=True) / denom
    cy = jnp.sum(Yr * Mr, keepdims=True) / denom
    cz = jnp.sum(Zr * Mr, keepdims=True) / denom

    tile = ptsN_ref[0, pl.ds(t * TM, TM), :]
    xi, yi, zi = tile[:, 0:1], tile[:, 1:2], tile[:, 2:3]

    lanes = lax.broadcasted_iota(jnp.int32, (1, 128), 1)
    kmask = lanes < K
    dxk = jnp.where(kmask, nx_ref[0] - xi, 0.0)
    dyk = jnp.where(kmask, ny_ref[0] - yi, 0.0)
    dzk = jnp.where(kmask, nz_ref[0] - zi, 0.0)
    Kf = float(K)
    cxx = jnp.sum(dxk * dxk, axis=1, keepdims=True) / Kf
    cyy = jnp.sum(dyk * dyk, axis=1, keepdims=True) / Kf
    czz = jnp.sum(dzk * dzk, axis=1, keepdims=True) / Kf
    cxy = jnp.sum(dxk * dyk, axis=1, keepdims=True) / Kf
    cxz = jnp.sum(dxk * dzk, axis=1, keepdims=True) / Kf
    cyz = jnp.sum(dyk * dzk, axis=1, keepdims=True) / Kf

    q = (cxx + cyy + czz) / 3.0
    p1 = cxy * cxy + cxz * cxz + cyz * cyz
    dxx = cxx - q
    dyy = cyy - q
    dzz = czz - q
    p2 = dxx * dxx + dyy * dyy + dzz * dzz + 2.0 * p1
    degen = p2 <= 1e-22
    p = jnp.sqrt(jnp.maximum(p2, 1e-22) / 6.0)
    bxx = dxx / p
    byy = dyy / p
    bzz = dzz / p
    bxy = cxy / p
    bxz = cxz / p
    byz = cyz / p
    detb = (bxx * (byy * bzz - byz * byz)
            - bxy * (bxy * bzz - byz * bxz)
            + bxz * (bxy * byz - byy * bxz))
    r = jnp.clip(detb * 0.5, -1.0, 1.0)
    phi = jnp.arctan2(jnp.sqrt(jnp.maximum(1.0 - r * r, 0.0)), r) / 3.0
    ev2 = q + 2.0 * p * jnp.cos(phi)
    ev0 = q + 2.0 * p * jnp.cos(phi + 2.0 * jnp.pi / 3.0)
    ev2 = jnp.where(degen, q, ev2)
    ev0 = jnp.where(degen, q, ev0)
    curv = ev0 / (ev2 + 1e-8)

    dxc = xi - cx
    dyc = yi - cy
    dzc = zi - cz
    dist_c = jnp.sqrt(dxc * dxc + dyc * dyc + dzc * dzc)
    hdist = jnp.sqrt(dxc * dxc + dyc * dyc)
    rad = jnp.arctan2(dyc, dxc)

    gate = (msum > 0.0).astype(jnp.float32)
    w1b = w1b_ref[...]
    gcon = (dist_c * w1b[0:1, :] + dzc * w1b[1:2, :]
            + hdist * w1b[2:3, :] + curv * w1b[4:5, :]
            + rad * w1b[5:6, :]) * gate

    h = jnp.maximum(h1_ref[0] + gcon + b1_ref[...], 0.0)
    out = jnp.dot(h, w2_ref[...], preferred_element_type=jnp.float32)
    o_ref[0] = jnp.maximum(out + b2_ref[...], 0.0)


@jax.jit
def kernel(points, features, leaf_mask, W1, b1, W2, b2):
    maskf = leaf_mask.astype(jnp.float32)
    ptsT = jnp.concatenate(
        [points.transpose(0, 2, 1), maskf[:, None, :],
         jnp.zeros((B, 4, N), jnp.float32)], axis=1)
    ptsN = jnp.pad(points, ((0, 0), (0, 0), (0, 5)))
    w1a = W1[:D_IN, :]
    w1b = jnp.pad(W1[D_IN:, :], ((0, 2), (0, 0)))
    b1r = b1[None, :]
    b2r = b2[None, :]

    grid = (B, N // TM)
    d2, h1, thr = pl.pallas_call(
        _s1_body,
        grid=grid,
        in_specs=[
            pl.BlockSpec((1, 8, N), lambda b, t: (b, 0, 0)),
            pl.BlockSpec((1, N, 8), lambda b, t: (b, 0, 0)),
            pl.BlockSpec((1, TM, D_IN), lambda b, t: (b, t, 0)),
            pl.BlockSpec((D_IN, D_OUT), lambda b, t: (0, 0)),
            pl.BlockSpec((8, D_OUT), lambda b, t: (0, 0)),
        ],
        out_specs=[
            pl.BlockSpec((1, TM, N), lambda b, t: (b, t, 0)),
            pl.BlockSpec((1, TM, D_OUT), lambda b, t: (b, t, 0)),
            pl.BlockSpec((1, TM, 1), lambda b, t: (b, t, 0)),
        ],
        out_shape=[
            jax.ShapeDtypeStruct((B, N, N), jnp.float32),
            jax.ShapeDtypeStruct((B, N, D_OUT), jnp.float32),
            jax.ShapeDtypeStruct((B, N, 1), jnp.float32),
        ],
    )(ptsT, ptsN, features, w1a, w1b)

    d2f = d2.reshape(B * N, N)
    px = points[..., 0].reshape(B * N)
    py = points[..., 1].reshape(B * N)
    pz = points[..., 2].reshape(B * N)
    nx, ny, nz = _sc_topk(d2f, px, py, pz, thr.reshape(B * N))
    nx = nx.reshape(B, N, 128)
    ny = ny.reshape(B, N, 128)
    nz = nz.reshape(B, N, 128)

    return pl.pallas_call(
        _s3_body,
        grid=grid,
        in_specs=[
            pl.BlockSpec((1, 8, N), lambda b, t: (b, 0, 0)),
            pl.BlockSpec((1, N, 8), lambda b, t: (b, 0, 0)),
            pl.BlockSpec((1, TM, 128), lambda b, t: (b, t, 0)),
            pl.BlockSpec((1, TM, 128), lambda b, t: (b, t, 0)),
            pl.BlockSpec((1, TM, 128), lambda b, t: (b, t, 0)),
            pl.BlockSpec((1, TM, D_OUT), lambda b, t: (b, t, 0)),
            pl.BlockSpec((8, D_OUT), lambda b, t: (0, 0)),
            pl.BlockSpec((1, D_OUT), lambda b, t: (0, 0)),
            pl.BlockSpec((D_OUT, D_OUT), lambda b, t: (0, 0)),
            pl.BlockSpec((1, D_OUT), lambda b, t: (0, 0)),
        ],
        out_specs=pl.BlockSpec((1, TM, D_OUT), lambda b, t: (b, t, 0)),
        out_shape=jax.ShapeDtypeStruct((B, N, D_OUT), jnp.float32),
    )(ptsT, ptsN, nx, ny, nz, h1, w1b, b1r, W2, b2r)


# speedup vs baseline: 4.3425x; 1.3532x over previous
import functools

import jax
import jax.numpy as jnp
from jax import lax
from jax.experimental import pallas as pl
from jax.experimental.pallas import tpu as pltpu
from jax.experimental.pallas import tpu_sc as plsc

B, N, D_IN, D_OUT, K = 2, 4096, 128, 256, 10
TM = 256
NT = 2048
NSC = N - NT
T_TC = NT // TM
NC, NS, L = 2, 16, 16
NW = NC * NS
RPW = (B * NSC) // NW
NCHUNK = N // L
RPD = 8
SB = 2 * RPD
GC = 8
_INF = 1e30


def _geom_mlp(xi, yi, zi, cxx, cyy, czz, cxy, cxz, cyz, cx, cy, cz, msum,
              w1b, h1, b1, w2, b2):
    q = (cxx + cyy + czz) / 3.0
    p1 = cxy * cxy + cxz * cxz + cyz * cyz
    dxx = cxx - q
    dyy = cyy - q
    dzz = czz - q
    p2 = dxx * dxx + dyy * dyy + dzz * dzz + 2.0 * p1
    degen = p2 <= 1e-22
    p = jnp.sqrt(jnp.maximum(p2, 1e-22) / 6.0)
    bxx = dxx / p
    byy = dyy / p
    bzz = dzz / p
    bxy = cxy / p
    bxz = cxz / p
    byz = cyz / p
    detb = (bxx * (byy * bzz - byz * byz)
            - bxy * (bxy * bzz - byz * bxz)
            + bxz * (bxy * byz - byy * bxz))
    r = jnp.clip(detb * 0.5, -1.0, 1.0)
    phi = jnp.arctan2(jnp.sqrt(jnp.maximum(1.0 - r * r, 0.0)), r) / 3.0
    ev2 = q + 2.0 * p * jnp.cos(phi)
    ev0 = q + 2.0 * p * jnp.cos(phi + 2.0 * jnp.pi / 3.0)
    ev2 = jnp.where(degen, q, ev2)
    ev0 = jnp.where(degen, q, ev0)
    curv = ev0 / (ev2 + 1e-8)

    dxc = xi - cx
    dyc = yi - cy
    dzc = zi - cz
    dist_c = jnp.sqrt(dxc * dxc + dyc * dyc + dzc * dzc)
    hdist = jnp.sqrt(dxc * dxc + dyc * dyc)
    rad = jnp.arctan2(dyc, dxc)

    gate = (msum > 0.0).astype(jnp.float32)
    gcon = (dist_c * w1b[0:1, :] + dzc * w1b[1:2, :]
            + hdist * w1b[2:3, :] + curv * w1b[4:5, :]
            + rad * w1b[5:6, :]) * gate
    h = jnp.maximum(h1 + gcon + b1, 0.0)
    out = jnp.dot(h, w2, preferred_element_type=jnp.float32)
    return jnp.maximum(out + b2, 0.0)


def _centroid(PT):
    Xr, Yr, Zr, Mr = PT[0:1, :], PT[1:2, :], PT[2:3, :], PT[3:4, :]
    msum = jnp.sum(Mr, keepdims=True)
    denom = jnp.maximum(msum, 1.0)
    cx = jnp.sum(Xr * Mr, keepdims=True) / denom
    cy = jnp.sum(Yr * Mr, keepdims=True) / denom
    cz = jnp.sum(Zr * Mr, keepdims=True) / denom
    return Xr, Yr, Zr, Mr, msum, cx, cy, cz


def _s1_body(ptsT_ref, ptsN_ref, f_ref, w1a_ref, w1b_ref, d2_ref, h1_ref,
             thr_ref):
    t = pl.program_id(1)
    PT = ptsT_ref[0]
    Mr = PT[3:4, :]
    msum = jnp.sum(Mr, keepdims=True)
    tile = ptsN_ref[0, pl.ds(t * TM, TM), :]
    xi, yi, zi = tile[:, 0:1], tile[:, 1:2], tile[:, 2:3]
    dx = xi - PT[0:1, :]
    dy = yi - PT[1:2, :]
    dz = zi - PT[2:3, :]
    d2 = (dx * dx + dy * dy) + dz * dz
    d2_ref[0] = d2
    density = jnp.sum(
        jnp.where((d2 < (0.02 ** 2)) & (Mr > 0.0), 1.0, 0.0),
        axis=1, keepdims=True)
    gate = (msum > 0.0).astype(jnp.float32)
    h = jnp.dot(f_ref[0], w1a_ref[...], preferred_element_type=jnp.float32)
    h1_ref[0] = h + (density * gate) * w1b_ref[3:4, :]
    bm = d2[:, 0:128]
    for kb in range(1, N // 128):
        bm = jnp.minimum(bm, d2[:, kb * 128:(kb + 1) * 128])
    m = jnp.min(bm, axis=1, keepdims=True)
    for _ in range(K - 1):
        bm = jnp.where(bm == m, _INF, bm)
        m = jnp.min(bm, axis=1, keepdims=True)
    thr_ref[0] = m


_mesh = plsc.VectorSubcoreMesh(core_axis_name="c", subcore_axis_name="s")


@functools.partial(
    pl.kernel,
    out_type=[jax.ShapeDtypeStruct((B * NSC, 128), jnp.float32)] * 3,
    mesh=_mesh,
    compiler_params=pltpu.CompilerParams(needs_layout_passes=False),
    scratch_types=[
        pltpu.VMEM((N,), jnp.float32),
        pltpu.VMEM((N,), jnp.float32),
        pltpu.VMEM((N,), jnp.float32),
        pltpu.VMEM((RPD, N), jnp.float32),
        pltpu.VMEM((RPD, N), jnp.float32),
        pltpu.VMEM((SB, 128), jnp.float32),
        pltpu.VMEM((SB, 128), jnp.float32),
        pltpu.VMEM((SB, 128), jnp.float32),
        pltpu.VMEM((RPW,), jnp.float32),
        pltpu.SemaphoreType.DMA,
        pltpu.SemaphoreType.DMA,
    ],
)
def _sc_topk(d2_hbm, px_hbm, py_hbm, pz_hbm, thr_hbm, nx_hbm, ny_hbm, nz_hbm,
             xv, yv, zv, rowa, rowb, nxs, nys, nzs, thrv, sema, semb):
    cid = lax.axis_index("c")
    sid = lax.axis_index("s")
    w = sid * NC + cid
    g0 = w * RPW
    b = g0 // NSC
    pltpu.sync_copy(px_hbm.at[pl.ds(b * N, N)], xv)
    pltpu.sync_copy(py_hbm.at[pl.ds(b * N, N)], yv)
    pltpu.sync_copy(pz_hbm.at[pl.ds(b * N, N)], zv)
    pltpu.sync_copy(thr_hbm.at[pl.ds(g0, RPW)], thrv)

    iota = lax.broadcasted_iota(jnp.int32, (L,), 0)

    def do_row(buf, j, r, ts):
        thr = jnp.full((L,), ts, jnp.float32)

        def grp_fn(g, carry):
            base = g * (GC * L)
            m_or = None
            for u in range(GC):
                m = buf[j, pl.ds(base + u * L, L)] <= thr
                m_or = m if m_or is None else (m_or | m)

            def hit(c1):
                bk, bi = c1
                for u in range(GC):
                    v = buf[j, pl.ds(base + u * L, L)]

                    def merge(c2, v=v, u=u):
                        bk2, bi2 = c2
                        si = base + u * L + iota
                        sk, sv = plsc.sort_key_val(v, si)
                        rk = lax.rev(sk, (0,))
                        rv = lax.rev(sv, (0,))
                        tb = bk2 <= rk
                        mk = jnp.where(tb, bk2, rk)
                        mi = jnp.where(tb, bi2, rv)
                        nk, ni = plsc.sort_key_val(mk, mi)
                        return (nk, ni)

                    bk, bi = lax.cond(jnp.any(v <= thr), merge,
                                      lambda c2: c2, (bk, bi))
                return bk, bi

            return lax.cond(jnp.any(m_or), hit, lambda c1: c1, carry)

        bk, bi = lax.fori_loop(
            0, NCHUNK // GC, grp_fn,
            (jnp.full((L,), _INF, jnp.float32), jnp.zeros((L,), jnp.int32)))
        nxs[r, 0:L] = plsc.load_gather(xv, [bi])
        nys[r, 0:L] = plsc.load_gather(yv, [bi])
        nzs[r, 0:L] = plsc.load_gather(zv, [bi])

    pltpu.async_copy(d2_hbm.at[pl.ds(g0, RPD)], rowa, sema)

    def pair_fn(p, carry):
        g = g0 + SB * p
        tch = thrv[pl.ds(SB * p, L)]
        pltpu.async_copy(d2_hbm.at[pl.ds(g + RPD, RPD)], rowb, semb)
        pltpu.make_async_copy(d2_hbm.at[pl.ds(g, RPD)], rowa, sema).wait()
        for j in range(RPD):
            do_row(rowa, j, j, tch[j])

        @pl.when(p < RPW // SB - 1)
        def _():
            pltpu.async_copy(d2_hbm.at[pl.ds(g + SB, RPD)], rowa, sema)

        pltpu.make_async_copy(d2_hbm.at[pl.ds(g + RPD, RPD)], rowb, semb).wait()
        for j in range(RPD):
            do_row(rowb, j, RPD + j, tch[RPD + j])
        pltpu.sync_copy(nxs, nx_hbm.at[pl.ds(g, SB)])
        pltpu.sync_copy(nys, ny_hbm.at[pl.ds(g, SB)])
        pltpu.sync_copy(nzs, nz_hbm.at[pl.ds(g, SB)])
        return carry

    lax.fori_loop(0, RPW // SB, pair_fn, 0)


def _s3tc_body(ptsT_ref, ptsN_ref, p16_ref, h1_ref, w1b_ref, b1_ref,
               w2_ref, b2_ref, o_ref):
    t = pl.program_id(1)
    PT = ptsT_ref[0]
    Xr, Yr, Zr, Mr, msum, cx, cy, cz = _centroid(PT)

    tile = ptsN_ref[0, pl.ds(t * TM, TM), :]
    xi, yi, zi = tile[:, 0:1], tile[:, 1:2], tile[:, 2:3]

    dx = xi - Xr
    dy = yi - Yr
    dz = zi - Zr
    d2 = (dx * dx + dy * dy) + dz * dz

    iota = lax.broadcasted_iota(jnp.int32, (1, N), 1)
    work = d2
    S = jnp.zeros((TM, N), jnp.float32)
    for _ in range(K):
        m = jnp.min(work, axis=1, keepdims=True)
        eq = work == m
        ji = jnp.min(jnp.where(eq, iota, N), axis=1, keepdims=True)
        sel = iota == ji
        S = S + sel.astype(jnp.float32)
        work = jnp.where(sel, _INF, work)

    M = jnp.dot(S, p16_ref[0], preferred_element_type=jnp.float32,
                precision=jax.lax.Precision.HIGHEST)
    m1x = M[:, 0:1]
    m1y = M[:, 1:2]
    m1z = M[:, 2:3]
    Kf = float(K)
    cxx = (M[:, 3:4] - 2.0 * xi * m1x + Kf * xi * xi) / Kf
    cyy = (M[:, 4:5] - 2.0 * yi * m1y + Kf * yi * yi) / Kf
    czz = (M[:, 5:6] - 2.0 * zi * m1z + Kf * zi * zi) / Kf
    cxy = (M[:, 6:7] - xi * m1y - yi * m1x + Kf * xi * yi) / Kf
    cxz = (M[:, 7:8] - xi * m1z - zi * m1x + Kf * xi * zi) / Kf
    cyz = (M[:, 8:9] - yi * m1z - zi * m1y + Kf * yi * zi) / Kf

    o_ref[0] = _geom_mlp(xi, yi, zi, cxx, cyy, czz, cxy, cxz, cyz,
                         cx, cy, cz, msum, w1b_ref[...], h1_ref[0],
                         b1_ref[...], w2_ref[...], b2_ref[...])


def _s3sc_body(ptsT_ref, ptsN_ref, nx_ref, ny_ref, nz_ref, h1_ref, w1b_ref,
               b1_ref, w2_ref, b2_ref, o_ref):
    t = pl.program_id(1)
    PT = ptsT_ref[0]
    Xr, Yr, Zr, Mr, msum, cx, cy, cz = _centroid(PT)

    tile = ptsN_ref[0, pl.ds(NT + t * TM, TM), :]
    xi, yi, zi = tile[:, 0:1], tile[:, 1:2], tile[:, 2:3]

    lanes = lax.broadcasted_iota(jnp.int32, (1, 128), 1)
    kmask = lanes < K
    dxk = jnp.where(kmask, nx_ref[0] - xi, 0.0)
    dyk = jnp.where(kmask, ny_ref[0] - yi, 0.0)
    dzk = jnp.where(kmask, nz_ref[0] - zi, 0.0)
    Kf = float(K)
    cxx = jnp.sum(dxk * dxk, axis=1, keepdims=True) / Kf
    cyy = jnp.sum(dyk * dyk, axis=1, keepdims=True) / Kf
    czz = jnp.sum(dzk * dzk, axis=1, keepdims=True) / Kf
    cxy = jnp.sum(dxk * dyk, axis=1, keepdims=True) / Kf
    cxz = jnp.sum(dxk * dzk, axis=1, keepdims=True) / Kf
    cyz = jnp.sum(dyk * dzk, axis=1, keepdims=True) / Kf

    o_ref[0] = _geom_mlp(xi, yi, zi, cxx, cyy, czz, cxy, cxz, cyz,
                         cx, cy, cz, msum, w1b_ref[...], h1_ref[0],
                         b1_ref[...], w2_ref[...], b2_ref[...])


@jax.jit
def kernel(points, features, leaf_mask, W1, b1, W2, b2):
    maskf = leaf_mask.astype(jnp.float32)
    ptsT = jnp.concatenate(
        [points.transpose(0, 2, 1), maskf[:, None, :],
         jnp.zeros((B, 4, N), jnp.float32)], axis=1)
    ptsN = jnp.pad(points, ((0, 0), (0, 0), (0, 5)))
    x, y, z = points[..., 0:1], points[..., 1:2], points[..., 2:3]
    p16 = jnp.concatenate(
        [x, y, z, x * x, y * y, z * z, x * y, x * z, y * z,
         jnp.zeros((B, N, 7), jnp.float32)], axis=-1)
    w1a = W1[:D_IN, :]
    w1b = jnp.pad(W1[D_IN:, :], ((0, 2), (0, 0)))
    b1r = b1[None, :]
    b2r = b2[None, :]

    grid1 = (B, N // TM)
    d2, h1, thr = pl.pallas_call(
        _s1_body,
        grid=grid1,
        in_specs=[
            pl.BlockSpec((1, 8, N), lambda b, t: (b, 0, 0)),
            pl.BlockSpec((1, N, 8), lambda b, t: (b, 0, 0)),
            pl.BlockSpec((1, TM, D_IN), lambda b, t: (b, t, 0)),
            pl.BlockSpec((D_IN, D_OUT), lambda b, t: (0, 0)),
            pl.BlockSpec((8, D_OUT), lambda b, t: (0, 0)),
        ],
        out_specs=[
            pl.BlockSpec((1, TM, N),
                         lambda b, t: (b, jnp.maximum(t - T_TC, 0), 0)),
            pl.BlockSpec((1, TM, D_OUT), lambda b, t: (b, t, 0)),
            pl.BlockSpec((1, TM, 1),
                         lambda b, t: (b, jnp.maximum(t - T_TC, 0), 0)),
        ],
        out_shape=[
            jax.ShapeDtypeStruct((B, NSC, N), jnp.float32),
            jax.ShapeDtypeStruct((B, N, D_OUT), jnp.float32),
            jax.ShapeDtypeStruct((B, NSC, 1), jnp.float32),
        ],
    )(ptsT, ptsN, features, w1a, w1b)

    d2f = d2.reshape(B * NSC, N)
    px = points[..., 0].reshape(B * N)
    py = points[..., 1].reshape(B * N)
    pz = points[..., 2].reshape(B * N)
    nx, ny, nz = _sc_topk(d2f, px, py, pz, thr.reshape(B * NSC))
    nx = nx.reshape(B, NSC, 128)
    ny = ny.reshape(B, NSC, 128)
    nz = nz.reshape(B, NSC, 128)

    out_tc = pl.pallas_call(
        _s3tc_body,
        grid=(B, T_TC),
        in_specs=[
            pl.BlockSpec((1, 8, N), lambda b, t: (b, 0, 0)),
            pl.BlockSpec((1, N, 8), lambda b, t: (b, 0, 0)),
            pl.BlockSpec((1, N, 16), lambda b, t: (b, 0, 0)),
            pl.BlockSpec((1, TM, D_OUT), lambda b, t: (b, t, 0)),
            pl.BlockSpec((8, D_OUT), lambda b, t: (0, 0)),
            pl.BlockSpec((1, D_OUT), lambda b, t: (0, 0)),
            pl.BlockSpec((D_OUT, D_OUT), lambda b, t: (0, 0)),
            pl.BlockSpec((1, D_OUT), lambda b, t: (0, 0)),
        ],
        out_specs=pl.BlockSpec((1, TM, D_OUT), lambda b, t: (b, t, 0)),
        out_shape=jax.ShapeDtypeStruct((B, NT, D_OUT), jnp.float32),
    )(ptsT, ptsN, p16, h1, w1b, b1r, W2, b2r)

    out_sc = pl.pallas_call(
        _s3sc_body,
        grid=(B, NSC // TM),
        in_specs=[
            pl.BlockSpec((1, 8, N), lambda b, t: (b, 0, 0)),
            pl.BlockSpec((1, N, 8), lambda b, t: (b, 0, 0)),
            pl.BlockSpec((1, TM, 128), lambda b, t: (b, t, 0)),
            pl.BlockSpec((1, TM, 128), lambda b, t: (b, t, 0)),
            pl.BlockSpec((1, TM, 128), lambda b, t: (b, t, 0)),
            pl.BlockSpec((1, TM, D_OUT), lambda b, t: (b, t + T_TC, 0)),
            pl.BlockSpec((8, D_OUT), lambda b, t: (0, 0)),
            pl.BlockSpec((1, D_OUT), lambda b, t: (0, 0)),
            pl.BlockSpec((D_OUT, D_OUT), lambda b, t: (0, 0)),
            pl.BlockSpec((1, D_OUT), lambda b, t: (0, 0)),
        ],
        out_specs=pl.BlockSpec((1, TM, D_OUT), lambda b, t: (b, t, 0)),
        out_shape=jax.ShapeDtypeStruct((B, NSC, D_OUT), jnp.float32),
    )(ptsT, ptsN, nx, ny, nz, h1, w1b, b1r, W2, b2r)

    return jnp.concatenate([out_tc, out_sc], axis=1)
